# baseline (speedup 1.0000x reference)
import functools

import jax
import jax.numpy as jnp
from jax import lax
from jax.experimental import pallas as pl
from jax.experimental.pallas import tpu as pltpu
from jax.experimental.pallas import tpu_sc as plsc

VOCAB = 1000000
DIM = 64
BATCH = 16384
NEG = 20
KP1 = NEG + 1

NC = 2
NS = 16
NW = NC * NS

EPW = BATCH // NW
C = 16
NCHUNK = EPW // C
CO = C * KP1

O_SPLIT = [(0, 128), (128, 128), (256, CO - 256)]

_mesh = plsc.VectorSubcoreMesh(core_axis_name="c", subcore_axis_name="s")


RB = 512
NRB = -(-VOCAB // RB)


def _tc_relayout_body(in_t_ref, out_t_ref, tbl_ref):
    x = jnp.concatenate([in_t_ref[...], out_t_ref[...]], axis=0)
    tbl_ref[...] = x.T


_tc_relayout = pl.pallas_call(
    _tc_relayout_body,
    grid=(NRB,),
    in_specs=[
        pl.BlockSpec((DIM, RB), lambda i: (0, i)),
        pl.BlockSpec((DIM, RB), lambda i: (0, i)),
    ],
    out_specs=pl.BlockSpec((RB, 128), lambda i: (i, 0)),
    out_shape=jax.ShapeDtypeStruct((VOCAB, 128), jnp.float32),
)


@functools.partial(
    pl.kernel,
    out_type=jax.ShapeDtypeStruct((BATCH * KP1,), jnp.float32),
    mesh=_mesh,
    compiler_params=pltpu.CompilerParams(needs_layout_passes=False),
    scratch_types=[
        pltpu.VMEM((2 * C,), jnp.int32),
        pltpu.VMEM((2 * CO,), jnp.int32),
        pltpu.VMEM((2 * C, 128), jnp.float32),
        pltpu.VMEM((2 * CO, 128), jnp.float32),
        pltpu.VMEM((2 * CO,), jnp.float32),
        pltpu.SemaphoreType.DMA,
        pltpu.SemaphoreType.DMA,
    ],
)
def _sc_scores(center_hbm, out_idx_hbm, tbl_hbm, scores_hbm,
               cidx2, oidx2, crow2, orow2, sco2, sem_g, sem_s):
    wid = lax.axis_index("s") * NC + lax.axis_index("c")
    lane = lax.broadcasted_iota(jnp.int32, (16,), 0)
    ebase = wid * EPW

    def gather_copies(q):
        p = (q & 1) * C
        po = (q & 1) * CO
        cps = [pltpu.make_async_copy(
            tbl_hbm.at[cidx2.at[pl.ds(p, C)]],
            crow2.at[pl.ds(p, C)], sem_g)]
        for off, n in O_SPLIT:
            cps.append(pltpu.make_async_copy(
                tbl_hbm.at[oidx2.at[pl.ds(po + off, n)]],
                orow2.at[pl.ds(po + off, n)], sem_g))
        return cps

    def copy_idx(q):
        p = (q & 1) * C
        po = (q & 1) * CO
        b = ebase + q * C
        pltpu.sync_copy(center_hbm.at[pl.ds(b, C)], cidx2.at[pl.ds(p, C)])
        pltpu.sync_copy(out_idx_hbm.at[pl.ds(b * KP1, CO)],
                        oidx2.at[pl.ds(po, CO)])

    def score_write(q):
        po = (q & 1) * CO
        sb = (ebase + q * C) * KP1
        return pltpu.make_async_copy(
            sco2.at[pl.ds(po, CO)], scores_hbm.at[pl.ds(sb, CO)], sem_s)

    copy_idx(0)
    copy_idx(1)
    for cp in gather_copies(0):
        cp.start()

    def chunk_body(g, carry):
        p = (g & 1) * C
        po = (g & 1) * CO

        for cp in gather_copies(g):
            cp.wait()

        @pl.when(g + 1 < NCHUNK)
        def _():
            for cp in gather_copies(g + 1):
                cp.start()

        @pl.when(g + 2 < NCHUNK)
        def _():
            copy_idx(g + 2)

        @pl.when(g >= 2)
        def _():
            score_write(g - 2).wait()

        crow = p + lane
        orow = po + lane * KP1
        acc = [jnp.zeros((16,), jnp.float32) for _ in range(KP1)]
        for db in range(8):
            d0 = db * 8
            c_regs = [
                plsc.load_gather(
                    crow2, [crow, jnp.full((16,), d0 + t, jnp.int32)])
                for t in range(8)
            ]
            for k in range(KP1):
                opos = orow + k
                a = acc[k]
                for t in range(8):
                    o = plsc.load_gather(
                        orow2, [opos, jnp.full((16,), 64 + d0 + t,
                                               jnp.int32)])
                    a = a + c_regs[t] * o
                acc[k] = a
        for k in range(KP1):
            sco2[pl.ds(po + k * C, 16)] = acc[k]

        score_write(g).start()
        return carry

    lax.fori_loop(0, NCHUNK, chunk_body, 0)
    score_write(NCHUNK - 2).wait()
    score_write(NCHUNK - 1).wait()


def _tc_loss_body(s_ref, o_ref):
    s = s_ref[...]
    rows, cols = s.shape
    r = lax.broadcasted_iota(jnp.int32, (rows, cols), 0)
    c = lax.broadcasted_iota(jnp.int32, (rows, cols), 1)
    p = r * cols + c
    is_pos = ((p // C) % KP1) == 0
    t = jnp.where(is_pos, s, -s)
    ls = jnp.minimum(t, 0.0) - jnp.log1p(jnp.exp(-jnp.abs(t)))
    o_ref[0, 0] = -jnp.sum(ls) / BATCH


_tc_loss = pl.pallas_call(
    _tc_loss_body,
    out_shape=jax.ShapeDtypeStruct((1, 1), jnp.float32),
    out_specs=pl.BlockSpec(memory_space=pltpu.SMEM),
)


def kernel(center, context, negatives, in_emb, out_emb):
    tbl = _tc_relayout(in_emb.T, out_emb.T)
    out_idx = jnp.concatenate([context[:, None], negatives], axis=1)
    out_idx = out_idx.reshape(BATCH * KP1)
    scores = _sc_scores(center, out_idx, tbl)
    loss = _tc_loss(scores.reshape(336, 1024))
    return loss.reshape(())


# speedup vs baseline: 2.1214x; 2.1214x over previous
import functools

import jax
import jax.numpy as jnp
from jax import lax
from jax.experimental import pallas as pl
from jax.experimental.pallas import tpu as pltpu
from jax.experimental.pallas import tpu_sc as plsc

VOCAB = 1000000
DIM = 64
BATCH = 16384
NEG = 20
KP1 = NEG + 1

NC = 2
NS = 16
NW = NC * NS

EPW = BATCH // NW
C = 16
NCHUNK = EPW // C
CO = C * KP1

O_SPLIT = [(0, 128), (128, 128), (256, CO - 256)]

_mesh = plsc.VectorSubcoreMesh(core_axis_name="c", subcore_axis_name="s")


RB = 4096
NRB = -(-VOCAB // RB)


def _tc_relayout_body(in_t_ref, out_t_ref, tbl_ref):
    x = jnp.concatenate([in_t_ref[...], out_t_ref[...]], axis=0)
    tbl_ref[...] = x.T


_tc_relayout = pl.pallas_call(
    _tc_relayout_body,
    grid=(NRB,),
    in_specs=[
        pl.BlockSpec((DIM, RB), lambda i: (0, i)),
        pl.BlockSpec((DIM, RB), lambda i: (0, i)),
    ],
    out_specs=pl.BlockSpec((RB, 128), lambda i: (i, 0)),
    out_shape=jax.ShapeDtypeStruct((VOCAB, 128), jnp.float32),
)


@functools.partial(
    pl.kernel,
    out_type=jax.ShapeDtypeStruct((BATCH * KP1,), jnp.float32),
    mesh=_mesh,
    compiler_params=pltpu.CompilerParams(needs_layout_passes=False),
    scratch_types=[
        pltpu.VMEM((2 * C,), jnp.int32),
        pltpu.VMEM((2 * CO,), jnp.int32),
        pltpu.VMEM((2 * C, 128), jnp.float32),
        pltpu.VMEM((2 * CO, 128), jnp.float32),
        pltpu.VMEM((2 * CO,), jnp.float32),
        pltpu.SemaphoreType.DMA,
        pltpu.SemaphoreType.DMA,
    ],
)
def _sc_scores(center_hbm, out_idx_hbm, tbl_hbm, scores_hbm,
               cidx2, oidx2, crow2, orow2, sco2, sem_g, sem_s):
    wid = lax.axis_index("s") * NC + lax.axis_index("c")
    lane = lax.broadcasted_iota(jnp.int32, (16,), 0)
    ebase = wid * EPW

    def gather_copies(q):
        p = (q & 1) * C
        po = (q & 1) * CO
        cps = [pltpu.make_async_copy(
            tbl_hbm.at[cidx2.at[pl.ds(p, C)]],
            crow2.at[pl.ds(p, C)], sem_g)]
        for off, n in O_SPLIT:
            cps.append(pltpu.make_async_copy(
                tbl_hbm.at[oidx2.at[pl.ds(po + off, n)]],
                orow2.at[pl.ds(po + off, n)], sem_g))
        return cps

    def copy_idx(q):
        p = (q & 1) * C
        po = (q & 1) * CO
        b = ebase + q * C
        pltpu.sync_copy(center_hbm.at[pl.ds(b, C)], cidx2.at[pl.ds(p, C)])
        pltpu.sync_copy(out_idx_hbm.at[pl.ds(b * KP1, CO)],
                        oidx2.at[pl.ds(po, CO)])

    def score_write(q):
        po = (q & 1) * CO
        sb = (ebase + q * C) * KP1
        return pltpu.make_async_copy(
            sco2.at[pl.ds(po, CO)], scores_hbm.at[pl.ds(sb, CO)], sem_s)

    copy_idx(0)
    copy_idx(1)
    for cp in gather_copies(0):
        cp.start()

    def chunk_body(g, carry):
        p = (g & 1) * C
        po = (g & 1) * CO

        for cp in gather_copies(g):
            cp.wait()

        @pl.when(g + 1 < NCHUNK)
        def _():
            for cp in gather_copies(g + 1):
                cp.start()

        @pl.when(g + 2 < NCHUNK)
        def _():
            copy_idx(g + 2)

        @pl.when(g >= 2)
        def _():
            score_write(g - 2).wait()

        crow = p + lane
        orow = po + lane * KP1
        acc = [jnp.zeros((16,), jnp.float32) for _ in range(KP1)]
        for db in range(8):
            d0 = db * 8
            c_regs = [
                plsc.load_gather(
                    crow2, [crow, jnp.full((16,), d0 + t, jnp.int32)])
                for t in range(8)
            ]
            for k in range(KP1):
                opos = orow + k
                a = acc[k]
                for t in range(8):
                    o = plsc.load_gather(
                        orow2, [opos, jnp.full((16,), 64 + d0 + t,
                                               jnp.int32)])
                    a = a + c_regs[t] * o
                acc[k] = a
        for k in range(KP1):
            sco2[pl.ds(po + k * C, 16)] = acc[k]

        score_write(g).start()
        return carry

    lax.fori_loop(0, NCHUNK, chunk_body, 0)
    score_write(NCHUNK - 2).wait()
    score_write(NCHUNK - 1).wait()


def _tc_loss_body(s_ref, o_ref):
    s = s_ref[...]
    rows, cols = s.shape
    r = lax.broadcasted_iota(jnp.int32, (rows, cols), 0)
    c = lax.broadcasted_iota(jnp.int32, (rows, cols), 1)
    p = r * cols + c
    is_pos = ((p // C) % KP1) == 0
    t = jnp.where(is_pos, s, -s)
    ls = jnp.minimum(t, 0.0) - jnp.log1p(jnp.exp(-jnp.abs(t)))
    o_ref[0, 0] = -jnp.sum(ls) / BATCH


_tc_loss = pl.pallas_call(
    _tc_loss_body,
    out_shape=jax.ShapeDtypeStruct((1, 1), jnp.float32),
    out_specs=pl.BlockSpec(memory_space=pltpu.SMEM),
)


def kernel(center, context, negatives, in_emb, out_emb):
    tbl = _tc_relayout(in_emb.T, out_emb.T)
    out_idx = jnp.concatenate([context[:, None], negatives], axis=1)
    out_idx = out_idx.reshape(BATCH * KP1)
    scores = _sc_scores(center, out_idx, tbl)
    loss = _tc_loss(scores.reshape(336, 1024))
    return loss.reshape(())


# speedup vs baseline: 2.9803x; 1.4049x over previous
import functools

import jax
import jax.numpy as jnp
from jax import lax
from jax.experimental import pallas as pl
from jax.experimental.pallas import tpu as pltpu
from jax.experimental.pallas import tpu_sc as plsc

VOCAB = 1000000
DIM = 64
BATCH = 16384
NEG = 20
KP1 = NEG + 1

NC = 2
NS = 16
NW = NC * NS

EPW = BATCH // NW
C = 16
NCHUNK = EPW // C
CO = C * KP1

O_SPLIT = [(0, 128), (128, 128), (256, CO - 256)]

_mesh = plsc.VectorSubcoreMesh(core_axis_name="c", subcore_axis_name="s")


RB = 4096
NRB = -(-VOCAB // RB)


def _tc_relayout_body(in_t_ref, out_t_ref, tbl_ref):
    x = jnp.concatenate([in_t_ref[...], out_t_ref[...]], axis=0)
    tbl_ref[...] = x.T


_tc_relayout = pl.pallas_call(
    _tc_relayout_body,
    grid=(NRB,),
    in_specs=[
        pl.BlockSpec((DIM, RB), lambda i: (0, i)),
        pl.BlockSpec((DIM, RB), lambda i: (0, i)),
    ],
    out_specs=pl.BlockSpec((RB, 128), lambda i: (i, 0)),
    out_shape=jax.ShapeDtypeStruct((VOCAB, 128), jnp.float32),
)


@functools.partial(
    pl.kernel,
    out_type=jax.ShapeDtypeStruct((BATCH * KP1,), jnp.float32),
    mesh=_mesh,
    compiler_params=pltpu.CompilerParams(needs_layout_passes=False),
    scratch_types=[
        pltpu.VMEM((2 * C,), jnp.int32),
        pltpu.VMEM((2 * CO,), jnp.int32),
        pltpu.VMEM((2 * C, 128), jnp.float32),
        pltpu.VMEM((2 * CO, 128), jnp.float32),
        pltpu.VMEM((2 * CO,), jnp.float32),
        pltpu.SemaphoreType.DMA,
        pltpu.SemaphoreType.DMA,
    ],
)
def _sc_scores(center_hbm, out_idx_hbm, tbl_hbm, scores_hbm,
               cidx2, oidx2, crow2, orow2, sco2, sem_g, sem_s):
    wid = lax.axis_index("s") * NC + lax.axis_index("c")
    lane = lax.broadcasted_iota(jnp.int32, (16,), 0)
    ebase = wid * EPW

    def gather_copies(q):
        p = (q & 1) * C
        po = (q & 1) * CO
        cps = [pltpu.make_async_copy(
            tbl_hbm.at[cidx2.at[pl.ds(p, C)]],
            crow2.at[pl.ds(p, C)], sem_g)]
        for off, n in O_SPLIT:
            cps.append(pltpu.make_async_copy(
                tbl_hbm.at[oidx2.at[pl.ds(po + off, n)]],
                orow2.at[pl.ds(po + off, n)], sem_g))
        return cps

    def copy_idx(q):
        p = (q & 1) * C
        po = (q & 1) * CO
        b = ebase + q * C
        pltpu.sync_copy(center_hbm.at[pl.ds(b, C)], cidx2.at[pl.ds(p, C)])
        pltpu.sync_copy(out_idx_hbm.at[pl.ds(b * KP1, CO)],
                        oidx2.at[pl.ds(po, CO)])

    def score_write(q):
        po = (q & 1) * CO
        sb = (ebase + q * C) * KP1
        return pltpu.make_async_copy(
            sco2.at[pl.ds(po, CO)], scores_hbm.at[pl.ds(sb, CO)], sem_s)

    copy_idx(0)
    copy_idx(1)
    for cp in gather_copies(0):
        cp.start()

    def chunk_body(g, carry):
        p = (g & 1) * C
        po = (g & 1) * CO

        for cp in gather_copies(g):
            cp.wait()

        @pl.when(g + 1 < NCHUNK)
        def _():
            for cp in gather_copies(g + 1):
                cp.start()

        @pl.when(g + 2 < NCHUNK)
        def _():
            copy_idx(g + 2)

        @pl.when(g >= 2)
        def _():
            score_write(g - 2).wait()

        crow = p + lane
        orow = po + lane * KP1
        acc = [jnp.zeros((16,), jnp.float32) for _ in range(KP1)]
        for db in range(8):
            d0 = db * 8
            cols = [(lane + (d0 + t)) & (DIM - 1) for t in range(8)]
            c_regs = [plsc.load_gather(crow2, [crow, cols[t]])
                      for t in range(8)]
            for k in range(KP1):
                opos = orow + k
                a = acc[k]
                for t in range(8):
                    o = plsc.load_gather(orow2, [opos, DIM + cols[t]])
                    a = a + c_regs[t] * o
                acc[k] = a
        for k in range(KP1):
            sco2[pl.ds(po + k * C, 16)] = acc[k]

        score_write(g).start()
        return carry

    lax.fori_loop(0, NCHUNK, chunk_body, 0)
    score_write(NCHUNK - 2).wait()
    score_write(NCHUNK - 1).wait()


def _tc_loss_body(s_ref, o_ref):
    s = s_ref[...]
    rows, cols = s.shape
    r = lax.broadcasted_iota(jnp.int32, (rows, cols), 0)
    c = lax.broadcasted_iota(jnp.int32, (rows, cols), 1)
    p = r * cols + c
    is_pos = ((p // C) % KP1) == 0
    t = jnp.where(is_pos, s, -s)
    ls = jnp.minimum(t, 0.0) - jnp.log1p(jnp.exp(-jnp.abs(t)))
    o_ref[0, 0] = -jnp.sum(ls) / BATCH


_tc_loss = pl.pallas_call(
    _tc_loss_body,
    out_shape=jax.ShapeDtypeStruct((1, 1), jnp.float32),
    out_specs=pl.BlockSpec(memory_space=pltpu.SMEM),
)


def kernel(center, context, negatives, in_emb, out_emb):
    tbl = _tc_relayout(in_emb.T, out_emb.T)
    out_idx = jnp.concatenate([context[:, None], negatives], axis=1)
    out_idx = out_idx.reshape(BATCH * KP1)
    scores = _sc_scores(center, out_idx, tbl)
    loss = _tc_loss(scores.reshape(336, 1024))
    return loss.reshape(())


# speedup vs baseline: 3.3147x; 1.1122x over previous
import functools

import jax
import jax.numpy as jnp
from jax import lax
from jax.experimental import pallas as pl
from jax.experimental.pallas import tpu as pltpu
from jax.experimental.pallas import tpu_sc as plsc

VOCAB = 1000000
DIM = 64
BATCH = 16384
NEG = 20
KP1 = NEG + 1

NC = 2
NS = 16
NW = NC * NS

EPW = BATCH // NW
C = 16
NCHUNK = EPW // C
CO = C * KP1

O_SPLIT = [(0, 128), (128, 128), (256, CO - 256)]

_mesh = plsc.VectorSubcoreMesh(core_axis_name="c", subcore_axis_name="s")


RB = 8192
NRB = -(-VOCAB // RB)


def _tc_relayout_body(in_t_ref, out_t_ref, tbl_ref):
    x = jnp.concatenate([in_t_ref[...], out_t_ref[...]], axis=0)
    tbl_ref[...] = x.T


_tc_relayout = pl.pallas_call(
    _tc_relayout_body,
    grid=(NRB,),
    in_specs=[
        pl.BlockSpec((DIM, RB), lambda i: (0, i)),
        pl.BlockSpec((DIM, RB), lambda i: (0, i)),
    ],
    out_specs=pl.BlockSpec((RB, 128), lambda i: (i, 0)),
    out_shape=jax.ShapeDtypeStruct((VOCAB, 128), jnp.float32),
)


@functools.partial(
    pl.kernel,
    out_type=jax.ShapeDtypeStruct((BATCH * KP1,), jnp.float32),
    mesh=_mesh,
    compiler_params=pltpu.CompilerParams(needs_layout_passes=False),
    scratch_types=[
        pltpu.VMEM((2 * C,), jnp.int32),
        pltpu.VMEM((2 * CO,), jnp.int32),
        pltpu.VMEM((2 * C, 128), jnp.float32),
        pltpu.VMEM((2 * CO, 128), jnp.float32),
        pltpu.VMEM((2 * CO,), jnp.float32),
        pltpu.SemaphoreType.DMA,
        pltpu.SemaphoreType.DMA,
    ],
)
def _sc_scores(center_hbm, out_idx_hbm, tbl_hbm, scores_hbm,
               cidx2, oidx2, crow2, orow2, sco2, sem_g, sem_s):
    wid = lax.axis_index("s") * NC + lax.axis_index("c")
    lane = lax.broadcasted_iota(jnp.int32, (16,), 0)
    ebase = wid * EPW

    def gather_copies(q):
        p = (q & 1) * C
        po = (q & 1) * CO
        cps = [pltpu.make_async_copy(
            tbl_hbm.at[cidx2.at[pl.ds(p, C)]],
            crow2.at[pl.ds(p, C)], sem_g)]
        for off, n in O_SPLIT:
            cps.append(pltpu.make_async_copy(
                tbl_hbm.at[oidx2.at[pl.ds(po + off, n)]],
                orow2.at[pl.ds(po + off, n)], sem_g))
        return cps

    def copy_idx(q):
        p = (q & 1) * C
        po = (q & 1) * CO
        b = ebase + q * C
        pltpu.sync_copy(center_hbm.at[pl.ds(b, C)], cidx2.at[pl.ds(p, C)])
        pltpu.sync_copy(out_idx_hbm.at[pl.ds(b * KP1, CO)],
                        oidx2.at[pl.ds(po, CO)])

    def score_write(q):
        po = (q & 1) * CO
        sb = (ebase + q * C) * KP1
        return pltpu.make_async_copy(
            sco2.at[pl.ds(po, CO)], scores_hbm.at[pl.ds(sb, CO)], sem_s)

    copy_idx(0)
    copy_idx(1)
    for cp in gather_copies(0):
        cp.start()

    def chunk_body(g, carry):
        p = (g & 1) * C
        po = (g & 1) * CO

        for cp in gather_copies(g):
            cp.wait()

        @pl.when(g + 1 < NCHUNK)
        def _():
            for cp in gather_copies(g + 1):
                cp.start()

        @pl.when(g + 2 < NCHUNK)
        def _():
            copy_idx(g + 2)

        @pl.when(g >= 2)
        def _():
            score_write(g - 2).wait()

        crow = p + lane
        orow = po + lane * KP1
        acc = [jnp.zeros((16,), jnp.float32) for _ in range(KP1)]
        for db in range(8):
            d0 = db * 8
            cols = [(lane + (d0 + t)) & (DIM - 1) for t in range(8)]
            c_regs = [plsc.load_gather(crow2, [crow, cols[t]])
                      for t in range(8)]
            for k in range(KP1):
                opos = orow + k
                a = acc[k]
                for t in range(8):
                    o = plsc.load_gather(orow2, [opos, DIM + cols[t]])
                    a = a + c_regs[t] * o
                acc[k] = a
        for k in range(KP1):
            sco2[pl.ds(po + k * C, 16)] = acc[k]

        score_write(g).start()
        return carry

    lax.fori_loop(0, NCHUNK, chunk_body, 0)
    score_write(NCHUNK - 2).wait()
    score_write(NCHUNK - 1).wait()


def _tc_loss_body(s_ref, o_ref):
    s = s_ref[...]
    rows, cols = s.shape
    r = lax.broadcasted_iota(jnp.int32, (rows, cols), 0)
    c = lax.broadcasted_iota(jnp.int32, (rows, cols), 1)
    p = r * cols + c
    is_pos = ((p // C) % KP1) == 0
    t = jnp.where(is_pos, s, -s)
    ls = jnp.minimum(t, 0.0) - jnp.log1p(jnp.exp(-jnp.abs(t)))
    o_ref[0, 0] = -jnp.sum(ls) / BATCH


_tc_loss = pl.pallas_call(
    _tc_loss_body,
    out_shape=jax.ShapeDtypeStruct((1, 1), jnp.float32),
    out_specs=pl.BlockSpec(memory_space=pltpu.SMEM),
)


def kernel(center, context, negatives, in_emb, out_emb):
    tbl = _tc_relayout(in_emb.T, out_emb.T)
    out_idx = jnp.concatenate([context[:, None], negatives], axis=1)
    out_idx = out_idx.reshape(BATCH * KP1)
    scores = _sc_scores(center, out_idx, tbl)
    loss = _tc_loss(scores.reshape(336, 1024))
    return loss.reshape(())


# speedup vs baseline: 3.3835x; 1.0208x over previous
import functools

import jax
import jax.numpy as jnp
from jax import lax
from jax.experimental import pallas as pl
from jax.experimental.pallas import tpu as pltpu
from jax.experimental.pallas import tpu_sc as plsc

VOCAB = 1000000
DIM = 64
BATCH = 16384
NEG = 20
KP1 = NEG + 1

NC = 2
NS = 16
NW = NC * NS

EPW = BATCH // NW
C = 16
NCHUNK = EPW // C
CO = C * KP1

O_SPLIT = [(0, 128), (128, 128), (256, CO - 256)]

_mesh = plsc.VectorSubcoreMesh(core_axis_name="c", subcore_axis_name="s")


RB = 16384
NRB = -(-VOCAB // RB)


def _tc_relayout_body(in_t_ref, out_t_ref, tbl_ref):
    x = jnp.concatenate([in_t_ref[...], out_t_ref[...]], axis=0)
    tbl_ref[...] = x.T


_tc_relayout = pl.pallas_call(
    _tc_relayout_body,
    grid=(NRB,),
    in_specs=[
        pl.BlockSpec((DIM, RB), lambda i: (0, i)),
        pl.BlockSpec((DIM, RB), lambda i: (0, i)),
    ],
    out_specs=pl.BlockSpec((RB, 128), lambda i: (i, 0)),
    out_shape=jax.ShapeDtypeStruct((VOCAB, 128), jnp.float32),
)


@functools.partial(
    pl.kernel,
    out_type=jax.ShapeDtypeStruct((BATCH * KP1,), jnp.float32),
    mesh=_mesh,
    compiler_params=pltpu.CompilerParams(needs_layout_passes=False),
    scratch_types=[
        pltpu.VMEM((2 * C,), jnp.int32),
        pltpu.VMEM((2 * CO,), jnp.int32),
        pltpu.VMEM((2 * C, 128), jnp.float32),
        pltpu.VMEM((2 * CO, 128), jnp.float32),
        pltpu.VMEM((2 * CO,), jnp.float32),
        pltpu.SemaphoreType.DMA,
        pltpu.SemaphoreType.DMA,
    ],
)
def _sc_scores(center_hbm, out_idx_hbm, tbl_hbm, scores_hbm,
               cidx2, oidx2, crow2, orow2, sco2, sem_g, sem_s):
    wid = lax.axis_index("s") * NC + lax.axis_index("c")
    lane = lax.broadcasted_iota(jnp.int32, (16,), 0)
    ebase = wid * EPW

    def gather_copies(q):
        p = (q & 1) * C
        po = (q & 1) * CO
        cps = [pltpu.make_async_copy(
            tbl_hbm.at[cidx2.at[pl.ds(p, C)]],
            crow2.at[pl.ds(p, C)], sem_g)]
        for off, n in O_SPLIT:
            cps.append(pltpu.make_async_copy(
                tbl_hbm.at[oidx2.at[pl.ds(po + off, n)]],
                orow2.at[pl.ds(po + off, n)], sem_g))
        return cps

    def copy_idx(q):
        p = (q & 1) * C
        po = (q & 1) * CO
        b = ebase + q * C
        pltpu.sync_copy(center_hbm.at[pl.ds(b, C)], cidx2.at[pl.ds(p, C)])
        pltpu.sync_copy(out_idx_hbm.at[pl.ds(b * KP1, CO)],
                        oidx2.at[pl.ds(po, CO)])

    def score_write(q):
        po = (q & 1) * CO
        sb = (ebase + q * C) * KP1
        return pltpu.make_async_copy(
            sco2.at[pl.ds(po, CO)], scores_hbm.at[pl.ds(sb, CO)], sem_s)

    copy_idx(0)
    copy_idx(1)
    for cp in gather_copies(0):
        cp.start()

    def chunk_body(g, carry):
        p = (g & 1) * C
        po = (g & 1) * CO

        for cp in gather_copies(g):
            cp.wait()

        @pl.when(g + 1 < NCHUNK)
        def _():
            for cp in gather_copies(g + 1):
                cp.start()

        @pl.when(g + 2 < NCHUNK)
        def _():
            copy_idx(g + 2)

        @pl.when(g >= 2)
        def _():
            score_write(g - 2).wait()

        crow = p + lane
        orow = po + lane * KP1
        acc = [jnp.zeros((16,), jnp.float32) for _ in range(KP1)]
        for db in range(8):
            d0 = db * 8
            cols = [(lane + (d0 + t)) & (DIM - 1) for t in range(8)]
            c_regs = [plsc.load_gather(crow2, [crow, cols[t]])
                      for t in range(8)]
            for k in range(KP1):
                opos = orow + k
                a = acc[k]
                for t in range(8):
                    o = plsc.load_gather(orow2, [opos, DIM + cols[t]])
                    a = a + c_regs[t] * o
                acc[k] = a
        for k in range(KP1):
            sco2[pl.ds(po + k * C, 16)] = acc[k]

        score_write(g).start()
        return carry

    lax.fori_loop(0, NCHUNK, chunk_body, 0)
    score_write(NCHUNK - 2).wait()
    score_write(NCHUNK - 1).wait()


def _tc_loss_body(s_ref, o_ref):
    s = s_ref[...]
    rows, cols = s.shape
    r = lax.broadcasted_iota(jnp.int32, (rows, cols), 0)
    c = lax.broadcasted_iota(jnp.int32, (rows, cols), 1)
    p = r * cols + c
    is_pos = ((p // C) % KP1) == 0
    t = jnp.where(is_pos, s, -s)
    ls = jnp.minimum(t, 0.0) - jnp.log1p(jnp.exp(-jnp.abs(t)))
    o_ref[0, 0] = -jnp.sum(ls) / BATCH


_tc_loss = pl.pallas_call(
    _tc_loss_body,
    out_shape=jax.ShapeDtypeStruct((1, 1), jnp.float32),
    out_specs=pl.BlockSpec(memory_space=pltpu.SMEM),
)


def kernel(center, context, negatives, in_emb, out_emb):
    tbl = _tc_relayout(in_emb.T, out_emb.T)
    out_idx = jnp.concatenate([context[:, None], negatives], axis=1)
    out_idx = out_idx.reshape(BATCH * KP1)
    scores = _sc_scores(center, out_idx, tbl)
    loss = _tc_loss(scores.reshape(336, 1024))
    return loss.reshape(())
